# baseline (speedup 1.0000x reference)
import functools

import jax
import jax.numpy as jnp
from jax import lax
from jax.experimental import pallas as pl
from jax.experimental.pallas import tpu as pltpu
from jax.experimental.pallas import tpu_sc as plsc

NUM_CLASSES = 1_000_000
HIDDEN = 64
BATCH = 16384

_NC = 2
_NS = 16
_NW = _NC * _NS

_B_PER_W = BATCH // _NW
_CHUNK = 128
_NCHUNK = _B_PER_W // _CHUNK
_PACK = 2
_PROWS = NUM_CLASSES // _PACK


@functools.partial(
    pl.kernel,
    out_type=jax.ShapeDtypeStruct((BATCH, HIDDEN), jnp.float32),
    mesh=plsc.VectorSubcoreMesh(core_axis_name="c", subcore_axis_name="s"),
    scratch_types=[
        pltpu.VMEM((_B_PER_W,), jnp.int32),
        pltpu.VMEM((_NCHUNK, _CHUNK), jnp.int32),
        pltpu.VMEM((_CHUNK, _PACK * HIDDEN), jnp.float32),
        pltpu.VMEM((_CHUNK, _PACK * HIDDEN), jnp.float32),
        pltpu.VMEM((_CHUNK, HIDDEN), jnp.float32),
        pltpu.SemaphoreType.DMA,
        pltpu.SemaphoreType.DMA,
    ],
)
def _gather_kernel(table_hbm, idx_hbm, out_hbm, labels_v, tidx_v, rowsA,
                   rowsB, rowbuf, semA, semB):
    wid = lax.axis_index("s") * _NC + lax.axis_index("c")
    base = wid * _B_PER_W
    pltpu.sync_copy(idx_hbm.at[pl.ds(base, _B_PER_W)], labels_v)

    for j in range(_NCHUNK):
        for t in range(_CHUNK // 16):
            lvec = labels_v[pl.ds(j * _CHUNK + 16 * t, 16)]
            tidx_v[j, pl.ds(16 * t, 16)] = lax.shift_right_logical(lvec, 1)

    def gather(j, rows, sem):
        return pltpu.async_copy(table_hbm.at[tidx_v.at[j]], rows, sem)

    def select(j, rows):
        for h in range(_CHUNK // 16):
            lvec = labels_v[pl.ds(j * _CHUNK + 16 * h, 16)]
            pvec = lax.bitwise_and(lvec, 1)
            for g in range(16):
                i = 16 * h + g
                hi = pvec[g] > 0
                for s in range(HIDDEN // 16):
                    lo_seg = rows[i, pl.ds(16 * s, 16)]
                    hi_seg = rows[i, pl.ds(HIDDEN + 16 * s, 16)]
                    rowbuf[i, pl.ds(16 * s, 16)] = jnp.where(
                        hi, hi_seg, lo_seg)
        pltpu.sync_copy(rowbuf, out_hbm.at[pl.ds(base + j * _CHUNK, _CHUNK)])

    cA = gather(0, rowsA, semA)
    cB = gather(1, rowsB, semB)
    cA.wait()
    select(0, rowsA)
    cA = gather(2, rowsA, semA)
    cB.wait()
    select(1, rowsB)
    cB = gather(3, rowsB, semB)
    cA.wait()
    select(2, rowsA)
    cB.wait()
    select(3, rowsB)


def kernel(labels, embedding_table):
    table2 = embedding_table.reshape(_PROWS, _PACK * HIDDEN)
    return _gather_kernel(table2, labels.astype(jnp.int32))


# speedup vs baseline: 1.7006x; 1.7006x over previous
import functools

import jax
import jax.numpy as jnp
from jax import lax
from jax.experimental import pallas as pl
from jax.experimental.pallas import tpu as pltpu
from jax.experimental.pallas import tpu_sc as plsc

NUM_CLASSES = 1_000_000
HIDDEN = 64
BATCH = 16384

_NC = 2
_NS = 16
_NW = _NC * _NS

_B_PER_W = BATCH // _NW
_CHUNK = 32
_NCHUNK = _B_PER_W // _CHUNK


@functools.partial(
    pl.kernel,
    out_type=jax.ShapeDtypeStruct((BATCH, HIDDEN), jnp.float32),
    mesh=plsc.VectorSubcoreMesh(core_axis_name="c", subcore_axis_name="s"),
    scratch_types=[
        pltpu.VMEM((_B_PER_W,), jnp.int32),
        pltpu.VMEM((_CHUNK, HIDDEN), jnp.float32),
        pltpu.SemaphoreType.DMA,
    ],
)
def _gather_kernel(table_hbm, idx_hbm, out_hbm, labels_v, rowbuf, sem):
    wid = lax.axis_index("s") * _NC + lax.axis_index("c")
    base = wid * _B_PER_W
    pltpu.sync_copy(idx_hbm.at[pl.ds(base, _B_PER_W)], labels_v)

    def do_chunk(j, _):
        copies = []
        for h in range(_CHUNK // 16):
            lvec = labels_v[pl.ds(j * _CHUNK + h * 16, 16)]
            for g in range(16):
                i = h * 16 + g
                lab = lvec[g]
                copies.append(
                    pltpu.async_copy(table_hbm.at[lab], rowbuf.at[i], sem))
        for c in copies:
            c.wait()
        pltpu.sync_copy(rowbuf, out_hbm.at[pl.ds(base + j * _CHUNK, _CHUNK)])
        return 0

    lax.fori_loop(0, _NCHUNK, do_chunk, 0)


def kernel(labels, embedding_table):
    return _gather_kernel(embedding_table, labels.astype(jnp.int32))


# speedup vs baseline: 1.7319x; 1.0184x over previous
import functools

import jax
import jax.numpy as jnp
from jax import lax
from jax.experimental import pallas as pl
from jax.experimental.pallas import tpu as pltpu
from jax.experimental.pallas import tpu_sc as plsc

NUM_CLASSES = 1_000_000
HIDDEN = 64
BATCH = 16384

_NC = 2
_NS = 16
_NW = _NC * _NS

_B_PER_W = BATCH // _NW


@functools.partial(
    pl.kernel,
    out_type=jax.ShapeDtypeStruct((BATCH, HIDDEN), jnp.float32),
    mesh=plsc.VectorSubcoreMesh(core_axis_name="c", subcore_axis_name="s"),
    scratch_types=[
        pltpu.VMEM((_B_PER_W,), jnp.int32),
        pltpu.VMEM((_B_PER_W, HIDDEN), jnp.float32),
        pltpu.SemaphoreType.DMA,
    ],
)
def _gather_kernel(table_hbm, idx_hbm, out_hbm, labels_v, rowbuf, sem):
    wid = lax.axis_index("s") * _NC + lax.axis_index("c")
    base = wid * _B_PER_W
    pltpu.sync_copy(idx_hbm.at[pl.ds(base, _B_PER_W)], labels_v)

    copies = []
    for h in range(_B_PER_W // 16):
        lvec = labels_v[pl.ds(h * 16, 16)]
        for g in range(16):
            i = h * 16 + g
            lab = lvec[g]
            copies.append(
                pltpu.async_copy(table_hbm.at[lab], rowbuf.at[i], sem))
    for c in copies:
        c.wait()
    pltpu.sync_copy(rowbuf, out_hbm.at[pl.ds(base, _B_PER_W)])


def kernel(labels, embedding_table):
    return _gather_kernel(embedding_table, labels.astype(jnp.int32))
